# baseline (speedup 1.0000x reference)
import jax
import jax.numpy as jnp
from jax import lax
from jax.experimental import pallas as pl
from jax.experimental.pallas import tpu as pltpu
from jax.experimental.pallas import tpu_sc as plsc

NC, NS, LANES = 2, 16, 16
NW = NC * NS
BB, LL = 4096, 200
CHUNK = 16
PER_W = BB // NW
NCHUNK = PER_W // CHUNK
NPT = CHUNK * LL
NGRP = NPT // LANES
NLG = (LL + LANES - 1) // LANES

OFF_WK, OFF_ST, OFF_PT, OFF_WM = 0, 21, 309, 336
TBL_PAD = 384


def _sc_body(wd_hbm, st_hbm, semO_hbm, ch0_hbm, ch1_hbm, ch2_hbm, ch3_hbm,
             ch4_hbm, spt_hbm, tbl_hbm, out_hbm, traj_hbm,
             ch0, ch1, ch2, ch3, ch4, spt_s, wd_s, st_s, semO_s, tbl_s,
             traj_s, out_s):
    wid = lax.axis_index("s") * NC + lax.axis_index("c")
    base = wid * PER_W
    iota = lax.iota(jnp.int32, LANES)

    pltpu.sync_copy(tbl_hbm, tbl_s)
    wsp = [[plsc.load_gather(tbl_s, [jnp.full((LANES,), OFF_WM + d * 8 + k,
                                              jnp.int32)])
            for k in range(8)] for d in range(3)]

    def chunk_body(ci, carry):
        b0 = base + ci * CHUNK
        pltpu.sync_copy(wd_hbm.at[pl.ds(b0, CHUNK)], wd_s)
        pltpu.sync_copy(st_hbm.at[pl.ds(b0, CHUNK)], st_s)
        pltpu.sync_copy(semO_hbm.at[pl.ds(b0 * 8, CHUNK * 8)], semO_s)
        pltpu.sync_copy(ch0_hbm.at[pl.ds(b0 * LL, NPT)], ch0)
        pltpu.sync_copy(ch1_hbm.at[pl.ds(b0 * LL, NPT)], ch1)
        pltpu.sync_copy(ch2_hbm.at[pl.ds(b0 * LL, NPT)], ch2)
        pltpu.sync_copy(ch3_hbm.at[pl.ds(b0 * LL, NPT)], ch3)
        pltpu.sync_copy(ch4_hbm.at[pl.ds(b0 * LL, NPT)], ch4)
        pltpu.sync_copy(spt_hbm.at[pl.ds(b0 * LL, NPT)], spt_s)

        wd16 = wd_s[...]
        st16 = st_s[...]
        comps = []
        for d in range(3):
            comps.append(plsc.load_gather(tbl_s, [wd16 * 3 + (OFF_WK + d)]))
        for d in range(6):
            comps.append(plsc.load_gather(tbl_s, [st16 * 6 + (OFF_ST + d)]))
        cols = [plsc.load_gather(semO_s, [iota * 8 + k]) for k in range(8)]
        for d in range(3):
            acc = cols[0] * wsp[d][0]
            for k in range(1, 8):
                acc = acc + cols[k] * wsp[d][k]
            comps.append(acc)
        for j in range(12):
            plsc.store_scatter(traj_s, [iota * 12 + j], comps[j])
        plsc.subcore_barrier()
        pltpu.sync_copy(traj_s, traj_hbm.at[pl.ds(b0 * 12, CHUNK * 12)])

        def grp_body(g, c2):
            p0 = g * LANES
            obase = (p0 + iota) * 20
            for c, ch in enumerate((ch0, ch1, ch2, ch3, ch4)):
                plsc.store_scatter(out_s, [obase + c], ch[pl.ds(p0, LANES)])
            sptv = spt_s[pl.ds(p0, LANES)]
            pt3 = sptv * 3 + OFF_PT
            for d in range(3):
                v = plsc.load_gather(tbl_s, [pt3 + d])
                plsc.store_scatter(out_s, [obase + (17 + d)], v)
            return c2
        lax.fori_loop(0, NGRP, grp_body, 0)

        for b in range(CHUNK):
            sel = iota == b
            tsp = [jnp.broadcast_to(
                       jnp.sum(jnp.where(sel, comps[j], 0.0)), (LANES,))
                   for j in range(12)]

            def bc_body(g, c3):
                lv = g * LANES + iota
                msk = lv < LL
                ob = b * (LL * 20) + jnp.minimum(lv, LL - 1) * 20
                for j in range(12):
                    plsc.store_scatter(out_s, [ob + (5 + j)], tsp[j],
                                       mask=msk)
                return c3
            lax.fori_loop(0, NLG, bc_body, 0)

        pltpu.sync_copy(out_s, out_hbm.at[pl.ds(b0 * (LL * 20), NPT * 20)])
        return carry

    lax.fori_loop(0, NCHUNK, chunk_body, 0)


def kernel(weekday, start_time, sem_O, lngs, lats, sem_pt, travel_dis, spd,
           azimuth, weekday_table, start_time_table, sem_pt_table, W_map):
    B, L = lngs.shape
    f32, i32 = jnp.float32, jnp.int32
    tbl = jnp.concatenate([
        weekday_table.reshape(-1), start_time_table.reshape(-1),
        sem_pt_table.reshape(-1), W_map.reshape(-1),
        jnp.zeros((TBL_PAD - 360,), f32),
    ])
    call = pl.kernel(
        _sc_body,
        out_type=[
            jax.ShapeDtypeStruct((B * L * 20,), f32),
            jax.ShapeDtypeStruct((B * 12,), f32),
        ],
        mesh=plsc.VectorSubcoreMesh(core_axis_name="c", subcore_axis_name="s",
                                    num_cores=NC, num_subcores=NS),
        compiler_params=pltpu.CompilerParams(needs_layout_passes=False),
        scratch_types=[
            pltpu.VMEM((NPT,), f32), pltpu.VMEM((NPT,), f32),
            pltpu.VMEM((NPT,), f32), pltpu.VMEM((NPT,), f32),
            pltpu.VMEM((NPT,), f32), pltpu.VMEM((NPT,), i32),
            pltpu.VMEM((CHUNK,), i32), pltpu.VMEM((CHUNK,), i32),
            pltpu.VMEM((CHUNK * 8,), f32), pltpu.VMEM((TBL_PAD,), f32),
            pltpu.VMEM((CHUNK * 12,), f32), pltpu.VMEM((NPT * 20,), f32),
        ],
    )
    out_flat, traj_flat = call(
        weekday.astype(i32), start_time.astype(i32), sem_O.reshape(-1),
        lngs.reshape(-1), lats.reshape(-1), travel_dis.reshape(-1),
        spd.reshape(-1), azimuth.reshape(-1),
        sem_pt.astype(i32).reshape(-1), tbl)
    return out_flat.reshape(B, L, 20), traj_flat.reshape(B, 12)


# speedup vs baseline: 1.0313x; 1.0313x over previous
import jax
import jax.numpy as jnp
from jax import lax
from jax.experimental import pallas as pl
from jax.experimental.pallas import tpu as pltpu
from jax.experimental.pallas import tpu_sc as plsc

NC, NS, LANES = 2, 16, 16
NW = NC * NS
BB, LL = 4096, 200
CHUNK = 16
PER_W = BB // NW
NCHUNK = PER_W // CHUNK
NPT = CHUNK * LL
NGRP = NPT // LANES
NLG = (LL + LANES - 1) // LANES

OFF_WK, OFF_ST, OFF_PT, OFF_WM = 0, 21, 309, 336
TBL_PAD = 384


def _sc_body(wd_hbm, st_hbm, semO_hbm, ch0_hbm, ch1_hbm, ch2_hbm, ch3_hbm,
             ch4_hbm, spt_hbm, tbl_hbm, out_hbm, traj_hbm,
             ch0, ch1, ch2, ch3, ch4, spt_s, wd_s, st_s, semO_s, tbl_s,
             traj_s, tcol_s, out_s):
    wid = lax.axis_index("s") * NC + lax.axis_index("c")
    base = wid * PER_W
    iota = lax.iota(jnp.int32, LANES)

    pltpu.sync_copy(tbl_hbm, tbl_s)
    wsp = [[plsc.load_gather(tbl_s, [jnp.full((LANES,), OFF_WM + d * 8 + k,
                                              jnp.int32)])
            for k in range(8)] for d in range(3)]

    def chunk_body(ci, carry):
        b0 = base + ci * CHUNK
        pltpu.sync_copy(wd_hbm.at[pl.ds(b0, CHUNK)], wd_s)
        pltpu.sync_copy(st_hbm.at[pl.ds(b0, CHUNK)], st_s)
        pltpu.sync_copy(semO_hbm.at[pl.ds(b0 * 8, CHUNK * 8)], semO_s)
        pltpu.sync_copy(ch0_hbm.at[pl.ds(b0 * LL, NPT)], ch0)
        pltpu.sync_copy(ch1_hbm.at[pl.ds(b0 * LL, NPT)], ch1)
        pltpu.sync_copy(ch2_hbm.at[pl.ds(b0 * LL, NPT)], ch2)
        pltpu.sync_copy(ch3_hbm.at[pl.ds(b0 * LL, NPT)], ch3)
        pltpu.sync_copy(ch4_hbm.at[pl.ds(b0 * LL, NPT)], ch4)
        pltpu.sync_copy(spt_hbm.at[pl.ds(b0 * LL, NPT)], spt_s)

        wd16 = wd_s[...]
        st16 = st_s[...]
        comps = []
        for d in range(3):
            comps.append(plsc.load_gather(tbl_s, [wd16 * 3 + (OFF_WK + d)]))
        for d in range(6):
            comps.append(plsc.load_gather(tbl_s, [st16 * 6 + (OFF_ST + d)]))
        cols = [plsc.load_gather(semO_s, [iota * 8 + k]) for k in range(8)]
        for d in range(3):
            acc = cols[0] * wsp[d][0]
            for k in range(1, 8):
                acc = acc + cols[k] * wsp[d][k]
            comps.append(acc)
        for j in range(12):
            plsc.store_scatter(traj_s, [iota * 12 + j], comps[j])
        plsc.subcore_barrier()
        pltpu.sync_copy(traj_s, traj_hbm.at[pl.ds(b0 * 12, CHUNK * 12)])

        @plsc.parallel_loop(0, NGRP, unroll=4)
        def grp_body(g):
            p0 = g * LANES
            obase = (p0 + iota) * 20
            for c, ch in enumerate((ch0, ch1, ch2, ch3, ch4)):
                plsc.store_scatter(out_s, [obase + c], ch[pl.ds(p0, LANES)])
            sptv = spt_s[pl.ds(p0, LANES)]
            pt3 = sptv * 3 + OFF_PT
            for d in range(3):
                v = plsc.load_gather(tbl_s, [pt3 + d])
                plsc.store_scatter(out_s, [obase + (17 + d)], v)

        for b in range(CHUNK):
            sel = iota == b
            tsp = [jnp.broadcast_to(
                       jnp.sum(jnp.where(sel, comps[j], 0.0)), (LANES,))
                   for j in range(12)]

            @plsc.parallel_loop(0, NLG, unroll=2)
            def bc_body(g):
                lv = g * LANES + iota
                msk = lv < LL
                ob = b * (LL * 20) + jnp.minimum(lv, LL - 1) * 20
                for j in range(12):
                    plsc.store_scatter(out_s, [ob + (5 + j)], tsp[j],
                                       mask=msk)

        pltpu.sync_copy(out_s, out_hbm.at[pl.ds(b0 * (LL * 20), NPT * 20)])
        return carry

    lax.fori_loop(0, NCHUNK, chunk_body, 0)


def kernel(weekday, start_time, sem_O, lngs, lats, sem_pt, travel_dis, spd,
           azimuth, weekday_table, start_time_table, sem_pt_table, W_map):
    B, L = lngs.shape
    f32, i32 = jnp.float32, jnp.int32
    tbl = jnp.concatenate([
        weekday_table.reshape(-1), start_time_table.reshape(-1),
        sem_pt_table.reshape(-1), W_map.reshape(-1),
        jnp.zeros((TBL_PAD - 360,), f32),
    ])
    call = pl.kernel(
        _sc_body,
        out_type=[
            jax.ShapeDtypeStruct((B * L * 20,), f32),
            jax.ShapeDtypeStruct((B * 12,), f32),
        ],
        mesh=plsc.VectorSubcoreMesh(core_axis_name="c", subcore_axis_name="s",
                                    num_cores=NC, num_subcores=NS),
        compiler_params=pltpu.CompilerParams(needs_layout_passes=False),
        scratch_types=[
            pltpu.VMEM((NPT,), f32), pltpu.VMEM((NPT,), f32),
            pltpu.VMEM((NPT,), f32), pltpu.VMEM((NPT,), f32),
            pltpu.VMEM((NPT,), f32), pltpu.VMEM((NPT,), i32),
            pltpu.VMEM((CHUNK,), i32), pltpu.VMEM((CHUNK,), i32),
            pltpu.VMEM((CHUNK * 8,), f32), pltpu.VMEM((TBL_PAD,), f32),
            pltpu.VMEM((CHUNK * 12,), f32), pltpu.VMEM((CHUNK * 12,), f32),
            pltpu.VMEM((NPT * 20,), f32),
        ],
    )
    out_flat, traj_flat = call(
        weekday.astype(i32), start_time.astype(i32), sem_O.reshape(-1),
        lngs.reshape(-1), lats.reshape(-1), travel_dis.reshape(-1),
        spd.reshape(-1), azimuth.reshape(-1),
        sem_pt.astype(i32).reshape(-1), tbl)
    return out_flat.reshape(B, L, 20), traj_flat.reshape(B, 12)


# speedup vs baseline: 1.0567x; 1.0247x over previous
import jax
import jax.numpy as jnp
from jax import lax
from jax.experimental import pallas as pl
from jax.experimental.pallas import tpu as pltpu
from jax.experimental.pallas import tpu_sc as plsc

NC, NS, LANES = 2, 16, 16
NW = NC * NS
BB, LL = 4096, 200
CHUNK = 16
PER_W = BB // NW
NCHUNK = PER_W // CHUNK
NPT = CHUNK * LL
NGRP = NPT // LANES
NLG = (LL + LANES - 1) // LANES

OFF_WK, OFF_ST, OFF_PT, OFF_WM = 0, 21, 309, 336
TBL_PAD = 384


def _sc_body(wd_hbm, st_hbm, semO_hbm, ch0_hbm, ch1_hbm, ch2_hbm, ch3_hbm,
             ch4_hbm, spt_hbm, tbl_hbm, out_hbm, traj_hbm,
             ch0, ch1, ch2, ch3, ch4, spt_s, wd_s, st_s, semO_s, tbl_s,
             traj_s, tcol_s, out_s, dsem):
    wid = lax.axis_index("s") * NC + lax.axis_index("c")
    base = wid * PER_W
    iota = lax.iota(jnp.int32, LANES)

    pltpu.sync_copy(tbl_hbm, tbl_s)
    wsp = [[plsc.load_gather(tbl_s, [jnp.full((LANES,), OFF_WM + d * 8 + k,
                                              jnp.int32)])
            for k in range(8)] for d in range(3)]

    def chunk_body(ci, carry):
        b0 = base + ci * CHUNK
        descs = [
            pltpu.async_copy(wd_hbm.at[pl.ds(b0, CHUNK)], wd_s, dsem),
            pltpu.async_copy(st_hbm.at[pl.ds(b0, CHUNK)], st_s, dsem),
            pltpu.async_copy(semO_hbm.at[pl.ds(b0 * 8, CHUNK * 8)], semO_s,
                             dsem),
            pltpu.async_copy(ch0_hbm.at[pl.ds(b0 * LL, NPT)], ch0, dsem),
            pltpu.async_copy(ch1_hbm.at[pl.ds(b0 * LL, NPT)], ch1, dsem),
            pltpu.async_copy(ch2_hbm.at[pl.ds(b0 * LL, NPT)], ch2, dsem),
            pltpu.async_copy(ch3_hbm.at[pl.ds(b0 * LL, NPT)], ch3, dsem),
            pltpu.async_copy(ch4_hbm.at[pl.ds(b0 * LL, NPT)], ch4, dsem),
            pltpu.async_copy(spt_hbm.at[pl.ds(b0 * LL, NPT)], spt_s, dsem),
        ]
        for d_ in descs:
            d_.wait()

        wd16 = wd_s[...]
        st16 = st_s[...]
        comps = []
        for d in range(3):
            comps.append(plsc.load_gather(tbl_s, [wd16 * 3 + (OFF_WK + d)]))
        for d in range(6):
            comps.append(plsc.load_gather(tbl_s, [st16 * 6 + (OFF_ST + d)]))
        cols = [plsc.load_gather(semO_s, [iota * 8 + k]) for k in range(8)]
        for d in range(3):
            acc = cols[0] * wsp[d][0]
            for k in range(1, 8):
                acc = acc + cols[k] * wsp[d][k]
            comps.append(acc)
        for j in range(12):
            plsc.store_scatter(traj_s, [iota * 12 + j], comps[j])
        plsc.subcore_barrier()
        pltpu.sync_copy(traj_s, traj_hbm.at[pl.ds(b0 * 12, CHUNK * 12)])

        @plsc.parallel_loop(0, NGRP, unroll=4)
        def grp_body(g):
            p0 = g * LANES
            obase = (p0 + iota) * 20
            for c, ch in enumerate((ch0, ch1, ch2, ch3, ch4)):
                plsc.store_scatter(out_s, [obase + c], ch[pl.ds(p0, LANES)])
            sptv = spt_s[pl.ds(p0, LANES)]
            pt3 = sptv * 3 + OFF_PT
            for d in range(3):
                v = plsc.load_gather(tbl_s, [pt3 + d])
                plsc.store_scatter(out_s, [obase + (17 + d)], v)

        for b in range(CHUNK):
            sel = iota == b
            tsp = [jnp.broadcast_to(
                       jnp.sum(jnp.where(sel, comps[j], 0.0)), (LANES,))
                   for j in range(12)]

            @plsc.parallel_loop(0, NLG, unroll=2)
            def bc_body(g):
                lv = g * LANES + iota
                msk = lv < LL
                ob = b * (LL * 20) + jnp.minimum(lv, LL - 1) * 20
                for j in range(12):
                    plsc.store_scatter(out_s, [ob + (5 + j)], tsp[j],
                                       mask=msk)

        pltpu.sync_copy(out_s, out_hbm.at[pl.ds(b0 * (LL * 20), NPT * 20)])
        return carry

    lax.fori_loop(0, NCHUNK, chunk_body, 0)


def kernel(weekday, start_time, sem_O, lngs, lats, sem_pt, travel_dis, spd,
           azimuth, weekday_table, start_time_table, sem_pt_table, W_map):
    B, L = lngs.shape
    f32, i32 = jnp.float32, jnp.int32
    tbl = jnp.concatenate([
        weekday_table.reshape(-1), start_time_table.reshape(-1),
        sem_pt_table.reshape(-1), W_map.reshape(-1),
        jnp.zeros((TBL_PAD - 360,), f32),
    ])
    call = pl.kernel(
        _sc_body,
        out_type=[
            jax.ShapeDtypeStruct((B * L * 20,), f32),
            jax.ShapeDtypeStruct((B * 12,), f32),
        ],
        mesh=plsc.VectorSubcoreMesh(core_axis_name="c", subcore_axis_name="s",
                                    num_cores=NC, num_subcores=NS),
        compiler_params=pltpu.CompilerParams(needs_layout_passes=False),
        scratch_types=[
            pltpu.VMEM((NPT,), f32), pltpu.VMEM((NPT,), f32),
            pltpu.VMEM((NPT,), f32), pltpu.VMEM((NPT,), f32),
            pltpu.VMEM((NPT,), f32), pltpu.VMEM((NPT,), i32),
            pltpu.VMEM((CHUNK,), i32), pltpu.VMEM((CHUNK,), i32),
            pltpu.VMEM((CHUNK * 8,), f32), pltpu.VMEM((TBL_PAD,), f32),
            pltpu.VMEM((CHUNK * 12,), f32), pltpu.VMEM((CHUNK * 12,), f32),
            pltpu.VMEM((NPT * 20,), f32), pltpu.SemaphoreType.DMA,
        ],
    )
    out_flat, traj_flat = call(
        weekday.astype(i32), start_time.astype(i32), sem_O.reshape(-1),
        lngs.reshape(-1), lats.reshape(-1), travel_dis.reshape(-1),
        spd.reshape(-1), azimuth.reshape(-1),
        sem_pt.astype(i32).reshape(-1), tbl)
    return out_flat.reshape(B, L, 20), traj_flat.reshape(B, 12)
